# baseline (speedup 1.0000x reference)
import functools

import jax
import jax.numpy as jnp
from jax import lax
from jax.experimental import pallas as pl
from jax.experimental.pallas import tpu as pltpu
from jax.experimental.pallas import tpu_sc as plsc

_NC = 2
_NS = 16
_L = 16


def _stats_body(x_ref, out_ref):
    i = pl.program_id(0)
    xb = x_ref[...]
    s1 = jnp.sum(xb, axis=0, keepdims=True)
    s2 = jnp.sum(xb * xb, axis=0, keepdims=True)
    blk = jnp.concatenate([s1, s2, jnp.zeros((6, xb.shape[1]), jnp.float32)], axis=0)

    @pl.when(i == 0)
    def _():
        out_ref[...] = blk

    @pl.when(i > 0)
    def _():
        out_ref[...] = out_ref[...] + blk


def _node_stats(x, bn):
    n, d = x.shape
    return pl.pallas_call(
        _stats_body,
        grid=(n // bn,),
        in_specs=[pl.BlockSpec((bn, d), lambda i: (i, 0))],
        out_specs=pl.BlockSpec((8, d), lambda i: (0, 0)),
        out_shape=jax.ShapeDtypeStruct((8, d), jnp.float32),
    )(x)


def _xform1_body(n, x_ref, st_ref, gnw_ref, gnb_ref, gnms_ref, wrel_ref, brel_ref,
                 wroot_ref, y_ref, r_ref):
    st = st_ref[...]
    mean = st[0:1, :] * (1.0 / n)
    ex2 = st[1:2, :] * (1.0 / n)
    mu = mean * gnms_ref[...]
    var = ex2 - 2.0 * mu * mean + mu * mu
    rstd = lax.rsqrt(var + 1e-5)
    xx = gnw_ref[...] * (x_ref[...] - mu) * rstd + gnb_ref[...]
    dn = (((1,), (1,)), ((), ()))
    y_ref[...] = lax.dot_general(xx, wrel_ref[...], dn,
                                 preferred_element_type=jnp.float32)
    r_ref[...] = lax.dot_general(xx, wroot_ref[...], dn,
                                 preferred_element_type=jnp.float32) + brel_ref[...]


def _xform1(x, stats, gnw, gnb, gnms, wrel, brel, wroot, bn):
    n, d = x.shape
    row = pl.BlockSpec((1, d), lambda i: (0, 0))
    full = pl.BlockSpec((d, d), lambda i: (0, 0))
    return pl.pallas_call(
        functools.partial(_xform1_body, n),
        grid=(n // bn,),
        in_specs=[
            pl.BlockSpec((bn, d), lambda i: (i, 0)),
            pl.BlockSpec((8, d), lambda i: (0, 0)),
            row, row, row, full, row, full,
        ],
        out_specs=[pl.BlockSpec((bn, d), lambda i: (i, 0))] * 2,
        out_shape=[jax.ShapeDtypeStruct((n, d), jnp.float32)] * 2,
    )(x, stats, gnw, gnb, gnms, wrel, brel, wroot)


def _xform2_body(a0_ref, a1_ref, r_ref, x_ref, wrel_ref, brel_ref, wroot_ref,
                 y_ref, r2_ref):
    h = a0_ref[...] + a1_ref[...] + r_ref[...]
    h = jnp.where(h > 0, h, 0.1 * h)
    dn = (((1,), (1,)), ((), ()))
    y_ref[...] = lax.dot_general(h, wrel_ref[...], dn,
                                 preferred_element_type=jnp.float32)
    r2_ref[...] = (lax.dot_general(h, wroot_ref[...], dn,
                                   preferred_element_type=jnp.float32)
                   + brel_ref[...] + x_ref[...])


def _xform2(a0, a1, r, x, wrel, brel, wroot, bn):
    n, d = x.shape
    row = pl.BlockSpec((1, d), lambda i: (0, 0))
    full = pl.BlockSpec((d, d), lambda i: (0, 0))
    blk = pl.BlockSpec((bn, d), lambda i: (i, 0))
    return pl.pallas_call(
        _xform2_body,
        grid=(n // bn,),
        in_specs=[blk, blk, blk, blk, full, row, full],
        out_specs=[blk] * 2,
        out_shape=[jax.ShapeDtypeStruct((n, d), jnp.float32)] * 2,
    )(a0, a1, r, x, wrel, brel, wroot)


def _final_body(a0_ref, a1_ref, r_ref, o_ref):
    o_ref[...] = a0_ref[...] + a1_ref[...] + r_ref[...]


def _final(a0, a1, r, bn):
    n, d = r.shape
    blk = pl.BlockSpec((bn, d), lambda i: (i, 0))
    return pl.pallas_call(
        _final_body,
        grid=(n // bn,),
        in_specs=[blk, blk, blk],
        out_specs=blk,
        out_shape=jax.ShapeDtypeStruct((n, d), jnp.float32),
    )(a0, a1, r)


def _sc_conv(y, src2, dst2, ew2, chunk):
    n, d = y.shape
    nrows, c = src2.shape
    nw = _NC * _NS
    tpw = nrows // nw
    zc = 80
    nzch = n // zc
    kmax = -(-nzch // _NS)

    mesh = plsc.VectorSubcoreMesh(core_axis_name="c", subcore_axis_name="s",
                                  num_cores=_NC, num_subcores=_NS)

    @functools.partial(
        pl.kernel,
        out_type=jax.ShapeDtypeStruct((_NC, n, d), jnp.float32),
        mesh=mesh,
        scratch_types=[
            pltpu.VMEM((tpw, c), jnp.int32),
            pltpu.VMEM((tpw, c), jnp.int32),
            pltpu.VMEM((tpw, c), jnp.float32),
            pltpu.VMEM((c, d), jnp.float32),
            pltpu.VMEM_SHARED((n, d), jnp.float32),
            pltpu.SemaphoreType.DMA,
        ],
    )
    def k(y_hbm, src_hbm, dst_hbm, ew_hbm, out_hbm, src_v, dst_v, ew_v, rows_v,
          agg_sh, sem):
        ci = lax.axis_index("c")
        si = lax.axis_index("s")
        w = si * _NC + ci

        def zrow(e, _):
            for j in range(d // _L):
                rows_v[e, pl.ds(j * _L, _L)] = jnp.zeros((_L,), jnp.float32)
            return 0

        lax.fori_loop(0, c, zrow, 0)

        def zchunk(k, _):
            t = k * _NS + si

            @pl.when(t < nzch)
            def _():
                pltpu.sync_copy(rows_v.at[pl.ds(0, zc)],
                                agg_sh.at[pl.ds(t * zc, zc)])
            return 0

        lax.fori_loop(0, kmax, zchunk, 0)
        plsc.subcore_barrier()

        pltpu.sync_copy(src_hbm.at[pl.ds(w * tpw, tpw)], src_v)
        pltpu.sync_copy(dst_hbm.at[pl.ds(w * tpw, tpw)], dst_v)
        pltpu.sync_copy(ew_hbm.at[pl.ds(w * tpw, tpw)], ew_v)

        def chunk_body(t, _):
            pltpu.async_copy(y_hbm.at[src_v.at[t]], rows_v, sem).wait()

            dnums = lax.GatherDimensionNumbers(
                offset_dims=(), collapsed_slice_dims=(0,), start_index_map=(0,))

            def scale(g, _):
                wv16 = ew_v[t, pl.ds(g * _L, _L)]
                for e in range(_L):
                    bc = lax.gather(
                        wv16, jnp.full((_L, 1), e, jnp.int32), dnums,
                        slice_sizes=(1,),
                        mode=lax.GatherScatterMode.PROMISE_IN_BOUNDS)
                    for j in range(d // _L):
                        rows_v[g * _L + e, pl.ds(j * _L, _L)] = (
                            rows_v[g * _L + e, pl.ds(j * _L, _L)] * bc)
                return 0

            pltpu.sync_copy(rows_v, agg_sh.at[dst_v.at[t]], add=True)
            return 0

        lax.fori_loop(0, tpw, chunk_body, 0)
        plsc.subcore_barrier()

        def dump(k, _):
            t = k * _NS + si

            @pl.when(t < nzch)
            def _():
                pltpu.sync_copy(agg_sh.at[pl.ds(t * zc, zc)],
                                out_hbm.at[ci, pl.ds(t * zc, zc)])
            return 0

        lax.fori_loop(0, kmax, dump, 0)

    return k(y, src2, dst2, ew2)


def kernel(x, edge_index, edge_weight, batch_map, gn_weight, gn_bias,
           gn_mean_scale, W_rel1, b_rel1, W_root1, W_rel2, b_rel2, W_root2):
    n, d = x.shape
    e = edge_weight.shape[0]
    chunk = 128
    bn = 2000

    nw = _NC * _NS
    ep = -(-e // (nw * 8 * chunk)) * (nw * 8 * chunk)
    pad = ep - e
    src1 = jnp.concatenate([edge_index[0], jnp.zeros((pad,), jnp.int32)])
    dst1 = jnp.concatenate([edge_index[1], jnp.zeros((pad,), jnp.int32)])
    ew1 = jnp.concatenate([edge_weight, jnp.zeros((pad,), jnp.float32)])
    src2 = src1.reshape(ep // chunk, chunk)
    dst2 = dst1.reshape(ep // chunk, chunk)
    ew2 = ew1.reshape(ep // chunk, chunk)
    gnw = gn_weight.reshape(1, d)
    gnb = gn_bias.reshape(1, d)
    gnms = gn_mean_scale.reshape(1, d)
    b1 = b_rel1.reshape(1, d)
    b2 = b_rel2.reshape(1, d)

    stats = _node_stats(x, bn)
    y1, r1 = _xform1(x, stats, gnw, gnb, gnms, W_rel1, b1, W_root1, bn)
    agg1 = _sc_conv(y1, src2, dst2, ew2, chunk)
    y2, r2 = _xform2(agg1[0], agg1[1], r1, x, W_rel2, b2, W_root2, bn)
    agg2 = _sc_conv(y2, src2, dst2, ew2, chunk)
    return _final(agg2[0], agg2[1], r2, bn)


# speedup vs baseline: 1.1967x; 1.1967x over previous
import functools

import jax
import jax.numpy as jnp
from jax import lax
from jax.experimental import pallas as pl
from jax.experimental.pallas import tpu as pltpu
from jax.experimental.pallas import tpu_sc as plsc

_NC = 2
_NS = 16
_L = 16
_NB = 2
_TB = 16


def _stats_body(x_ref, out_ref):
    i = pl.program_id(0)
    xb = x_ref[...]
    s1 = jnp.sum(xb, axis=0, keepdims=True)
    s2 = jnp.sum(xb * xb, axis=0, keepdims=True)
    blk = jnp.concatenate([s1, s2, jnp.zeros((6, xb.shape[1]), jnp.float32)], axis=0)

    @pl.when(i == 0)
    def _():
        out_ref[...] = blk

    @pl.when(i > 0)
    def _():
        out_ref[...] = out_ref[...] + blk


def _node_stats(x, bn):
    n, d = x.shape
    return pl.pallas_call(
        _stats_body,
        grid=(n // bn,),
        in_specs=[pl.BlockSpec((bn, d), lambda i: (i, 0))],
        out_specs=pl.BlockSpec((8, d), lambda i: (0, 0)),
        out_shape=jax.ShapeDtypeStruct((8, d), jnp.float32),
    )(x)


def _xform1_body(n, x_ref, st_ref, gnw_ref, gnb_ref, gnms_ref, wrel_ref, brel_ref,
                 wroot_ref, y_ref, r_ref):
    st = st_ref[...]
    mean = st[0:1, :] * (1.0 / n)
    ex2 = st[1:2, :] * (1.0 / n)
    mu = mean * gnms_ref[...]
    var = ex2 - 2.0 * mu * mean + mu * mu
    rstd = lax.rsqrt(var + 1e-5)
    xx = gnw_ref[...] * (x_ref[...] - mu) * rstd + gnb_ref[...]
    dn = (((1,), (1,)), ((), ()))
    y_ref[...] = lax.dot_general(xx, wrel_ref[...], dn,
                                 preferred_element_type=jnp.float32)
    r_ref[...] = lax.dot_general(xx, wroot_ref[...], dn,
                                 preferred_element_type=jnp.float32) + brel_ref[...]


def _xform1(x, stats, gnw, gnb, gnms, wrel, brel, wroot, bn):
    n, d = x.shape
    row = pl.BlockSpec((1, d), lambda i: (0, 0))
    full = pl.BlockSpec((d, d), lambda i: (0, 0))
    return pl.pallas_call(
        functools.partial(_xform1_body, n),
        grid=(n // bn,),
        in_specs=[
            pl.BlockSpec((bn, d), lambda i: (i, 0)),
            pl.BlockSpec((8, d), lambda i: (0, 0)),
            row, row, row, full, row, full,
        ],
        out_specs=[pl.BlockSpec((bn, d), lambda i: (i, 0))] * 2,
        out_shape=[jax.ShapeDtypeStruct((n, d), jnp.float32)] * 2,
    )(x, stats, gnw, gnb, gnms, wrel, brel, wroot)


def _xform2_body(a_ref, r_ref, x_ref, wrel_ref, brel_ref, wroot_ref,
                 y_ref, r2_ref):
    h = a_ref[0] + a_ref[1] + r_ref[...]
    h = jnp.where(h > 0, h, 0.1 * h)
    dn = (((1,), (1,)), ((), ()))
    y_ref[...] = lax.dot_general(h, wrel_ref[...], dn,
                                 preferred_element_type=jnp.float32)
    r2_ref[...] = (lax.dot_general(h, wroot_ref[...], dn,
                                   preferred_element_type=jnp.float32)
                   + brel_ref[...] + x_ref[...])


def _xform2(a, r, x, wrel, brel, wroot, bn):
    n, d = x.shape
    row = pl.BlockSpec((1, d), lambda i: (0, 0))
    full = pl.BlockSpec((d, d), lambda i: (0, 0))
    blk = pl.BlockSpec((bn, d), lambda i: (i, 0))
    return pl.pallas_call(
        _xform2_body,
        grid=(n // bn,),
        in_specs=[pl.BlockSpec((2, bn, d), lambda i: (0, i, 0)),
                  blk, blk, full, row, full],
        out_specs=[blk] * 2,
        out_shape=[jax.ShapeDtypeStruct((n, d), jnp.float32)] * 2,
    )(a, r, x, wrel, brel, wroot)


def _final_body(a_ref, r_ref, o_ref):
    o_ref[...] = a_ref[0] + a_ref[1] + r_ref[...]


def _final(a, r, bn):
    n, d = r.shape
    return pl.pallas_call(
        _final_body,
        grid=(n // bn,),
        in_specs=[pl.BlockSpec((2, bn, d), lambda i: (0, i, 0)),
                  pl.BlockSpec((bn, d), lambda i: (i, 0))],
        out_specs=pl.BlockSpec((bn, d), lambda i: (i, 0)),
        out_shape=jax.ShapeDtypeStruct((n, d), jnp.float32),
    )(a, r)


def _sc_conv(y, src2, dst2, ew2):
    n, d = y.shape
    nrows, c = src2.shape
    nw = _NC * _NS
    tpw = nrows // nw
    zc = 80
    nzch = n // zc
    kmax = -(-nzch // _NS)

    mesh = plsc.VectorSubcoreMesh(core_axis_name="c", subcore_axis_name="s",
                                  num_cores=_NC, num_subcores=_NS)

    @functools.partial(
        pl.kernel,
        out_type=jax.ShapeDtypeStruct((_NC, n, d), jnp.float32),
        mesh=mesh,
        scratch_types=[
            pltpu.VMEM((_TB, c), jnp.int32),
            pltpu.VMEM((_TB, c), jnp.int32),
            pltpu.VMEM((_TB, c), jnp.float32),
            [pltpu.VMEM((c, d), jnp.float32) for _ in range(_NB)],
            pltpu.VMEM_SHARED((n, d), jnp.float32),
            [pltpu.SemaphoreType.DMA for _ in range(_NB)],
        ],
    )
    def k(y_hbm, src_hbm, dst_hbm, ew_hbm, out_hbm, src_v, dst_v, ew_v, rows_v,
          agg_sh, sems):
        ci = lax.axis_index("c")
        si = lax.axis_index("s")
        w = si * _NC + ci

        def zrow(e, _):
            for j in range(d // _L):
                rows_v[0][e, pl.ds(j * _L, _L)] = jnp.zeros((_L,), jnp.float32)
            return 0

        lax.fori_loop(0, c, zrow, 0)

        def zchunk(kk, _):
            t = kk * _NS + si

            @pl.when(t < nzch)
            def _():
                pltpu.sync_copy(rows_v[0].at[pl.ds(0, zc)],
                                agg_sh.at[pl.ds(t * zc, zc)])
            return 0

        lax.fori_loop(0, kmax, zchunk, 0)
        plsc.subcore_barrier()

        dnums = lax.GatherDimensionNumbers(
            offset_dims=(), collapsed_slice_dims=(0,), start_index_map=(0,))

        def group_body(gi, _):
            base = w * tpw + gi * _TB
            pltpu.sync_copy(src_hbm.at[pl.ds(base, _TB)], src_v)
            pltpu.sync_copy(dst_hbm.at[pl.ds(base, _TB)], dst_v)
            pltpu.sync_copy(ew_hbm.at[pl.ds(base, _TB)], ew_v)
            for b in range(_NB):
                pltpu.async_copy(y_hbm.at[src_v.at[b]], rows_v[b], sems[b])

            def super_body(ss, _):
                for b in range(_NB):
                    lt = ss * _NB + b
                    pltpu.make_async_copy(y_hbm.at[src_v.at[lt]], rows_v[b],
                                          sems[b]).wait()

                    def scale(g, _):
                        wv16 = ew_v[lt, pl.ds(g * _L, _L)]
                        for e in range(_L):
                            bc = lax.gather(
                                wv16, jnp.full((_L, 1), e, jnp.int32), dnums,
                                slice_sizes=(1,),
                                mode=lax.GatherScatterMode.PROMISE_IN_BOUNDS)
                            for j in range(d // _L):
                                rows_v[b][g * _L + e, pl.ds(j * _L, _L)] = (
                                    rows_v[b][g * _L + e, pl.ds(j * _L, _L)]
                                    * bc)
                        return 0

                    lax.fori_loop(0, c // _L, scale, 0)
                    pltpu.sync_copy(rows_v[b], agg_sh.at[dst_v.at[lt]],
                                    add=True)

                    @pl.when(lt + _NB < _TB)
                    def _():
                        pltpu.async_copy(y_hbm.at[src_v.at[lt + _NB]],
                                         rows_v[b], sems[b])
                return 0

            lax.fori_loop(0, _TB // _NB, super_body, 0)
            return 0

        lax.fori_loop(0, tpw // _TB, group_body, 0)
        plsc.subcore_barrier()

        def dump(kk, _):
            t = kk * _NS + si

            @pl.when(t < nzch)
            def _():
                pltpu.sync_copy(agg_sh.at[pl.ds(t * zc, zc)],
                                out_hbm.at[ci, pl.ds(t * zc, zc)])
            return 0

        lax.fori_loop(0, kmax, dump, 0)

    return k(y, src2, dst2, ew2)


def kernel(x, edge_index, edge_weight, batch_map, gn_weight, gn_bias,
           gn_mean_scale, W_rel1, b_rel1, W_root1, W_rel2, b_rel2, W_root2):
    n, d = x.shape
    e = edge_weight.shape[0]
    chunk = 128
    bn = 2000

    nw = _NC * _NS
    ep = -(-e // (nw * 8 * chunk)) * (nw * 8 * chunk)
    pad = ep - e
    src1 = jnp.concatenate([edge_index[0], jnp.zeros((pad,), jnp.int32)])
    dst1 = jnp.concatenate([edge_index[1], jnp.zeros((pad,), jnp.int32)])
    ew1 = jnp.concatenate([edge_weight, jnp.zeros((pad,), jnp.float32)])
    src2 = src1.reshape(ep // chunk, chunk)
    dst2 = dst1.reshape(ep // chunk, chunk)
    ew2 = ew1.reshape(ep // chunk, chunk)
    gnw = gn_weight.reshape(1, d)
    gnb = gn_bias.reshape(1, d)
    gnms = gn_mean_scale.reshape(1, d)
    b1 = b_rel1.reshape(1, d)
    b2 = b_rel2.reshape(1, d)

    stats = _node_stats(x, bn)
    y1, r1 = _xform1(x, stats, gnw, gnb, gnms, W_rel1, b1, W_root1, bn)
    agg1 = _sc_conv(y1, src2, dst2, ew2)
    y2, r2 = _xform2(agg1, r1, x, W_rel2, b2, W_root2, bn)
    agg2 = _sc_conv(y2, src2, dst2, ew2)
    return _final(agg2, r2, bn)
